# Initial kernel scaffold; baseline (speedup 1.0000x reference)
import jax
import jax.numpy as jnp
from jax.experimental import pallas as pl


def kernel(int_feats, missing_mask, table, missing_emb, W1, b1, ln1_g, ln1_b, rms_s, W2, b2, ln2_g, ln2_b):
    raise NotImplementedError("write your pallas kernel here")


# speedup vs baseline: 1.2378x; 1.2378x over previous
import functools

import jax
import jax.numpy as jnp
from jax import lax
from jax.experimental import pallas as pl
from jax.experimental.pallas import tpu as pltpu
from jax.experimental.pallas import tpu_sc as plsc

B = 1024
F = 26
VOCAB = 1000
EMB = 128
DM = 512
NG = 13
NT = 8

NW = 32
PAIRS = B * NG
PW = PAIRS // NW
CHP = 32
NCHUNK = PW // CHP


def _sc_pool(table, ctab, ridx, cidx):
    mesh = plsc.VectorSubcoreMesh(core_axis_name="c", subcore_axis_name="s")

    @functools.partial(
        pl.kernel,
        mesh=mesh,
        out_type=jax.ShapeDtypeStruct((PAIRS, EMB), jnp.float32),
        scratch_types=[
            pltpu.VMEM((2 * CHP,), jnp.int32),
            pltpu.VMEM((CHP,), jnp.int32),
            pltpu.VMEM((2 * CHP, EMB), jnp.float32),
            pltpu.VMEM((CHP, EMB), jnp.float32),
            pltpu.VMEM((CHP, EMB), jnp.float32),
            pltpu.SemaphoreType.DMA,
            pltpu.SemaphoreType.DMA,
        ],
    )
    def k(table_h, ctab_h, ridx_h, cidx_h, out_h,
          ridx_v, cidx_v, rows_v, crows_v, out_v, sem1, sem2):
        wid = lax.axis_index("s") * 2 + lax.axis_index("c")

        def chunk(ci, _):
            pbase = wid * PW + ci * CHP
            pltpu.sync_copy(ridx_h.at[pl.ds(pbase * 2, 2 * CHP)], ridx_v)
            pltpu.sync_copy(cidx_h.at[pl.ds(pbase, CHP)], cidx_v)
            cp1 = pltpu.async_copy(table_h.at[ridx_v], rows_v, sem1)
            cp2 = pltpu.async_copy(ctab_h.at[cidx_v], crows_v, sem2)
            cp1.wait()
            cp2.wait()

            def pair(p, _2):
                for cc in range(EMB // 16):
                    s = pl.ds(cc * 16, 16)
                    out_v[p, s] = (rows_v[2 * p, s] + rows_v[2 * p + 1, s]) * 0.5 + crows_v[p, s]
                return 0

            lax.fori_loop(0, CHP, pair, 0)
            pltpu.sync_copy(out_v, out_h.at[pl.ds(pbase, CHP)])
            return 0

        lax.fori_loop(0, NCHUNK, chunk, 0)

    return k(table, ctab, ridx, cidx)


BT = 128


def _silu(x):
    return x / (1.0 + jnp.exp(-x))


def _tc_body(x3, w1, b1, g1, be1, rms3, w2, b2, g2, be2, outr):
    hs = []
    ss = jnp.zeros((BT, 1), jnp.float32)
    w1v = w1[...]
    for g in range(NG):
        h = jnp.dot(x3[g], w1v, preferred_element_type=jnp.float32)
        h = _silu(h + b1[...])
        mu = jnp.mean(h, axis=1, keepdims=True)
        hc = h - mu
        var = jnp.mean(hc * hc, axis=1, keepdims=True)
        h = hc * lax.rsqrt(var + 1e-5) * g1[...] + be1[...]
        hs.append(h)
        ss = ss + jnp.sum(h * h, axis=1, keepdims=True)
    inv = lax.rsqrt(ss * (1.0 / (NG * DM)) + 1e-6)
    acc = jnp.zeros((BT, NT * DM), jnp.float32)
    for g in range(NG):
        hn = hs[g] * inv * rms3[g]
        acc = acc + jnp.dot(hn.astype(jnp.bfloat16), w2[g],
                            preferred_element_type=jnp.float32)
    y = _silu(acc + b2[...])
    mu = jnp.mean(y, axis=1, keepdims=True)
    yc = y - mu
    var = jnp.mean(yc * yc, axis=1, keepdims=True)
    outr[...] = yc * lax.rsqrt(var + 1e-5) * g2[...] + be2[...]


def _tc_call(x3, w1, b1, g1, be1, rms3, w2bf, b2, g2, be2):
    return pl.pallas_call(
        _tc_body,
        grid=(B // BT,),
        in_specs=[
            pl.BlockSpec((NG, BT, EMB), lambda i: (0, i, 0)),
            pl.BlockSpec((EMB, DM), lambda i: (0, 0)),
            pl.BlockSpec((1, DM), lambda i: (0, 0)),
            pl.BlockSpec((1, DM), lambda i: (0, 0)),
            pl.BlockSpec((1, DM), lambda i: (0, 0)),
            pl.BlockSpec((NG, 1, DM), lambda i: (0, 0, 0)),
            pl.BlockSpec((NG, DM, NT * DM), lambda i: (0, 0, 0)),
            pl.BlockSpec((1, NT * DM), lambda i: (0, 0)),
            pl.BlockSpec((1, NT * DM), lambda i: (0, 0)),
            pl.BlockSpec((1, NT * DM), lambda i: (0, 0)),
        ],
        out_specs=pl.BlockSpec((BT, NT * DM), lambda i: (i, 0)),
        out_shape=jax.ShapeDtypeStruct((B, NT * DM), jnp.float32),
        compiler_params=pltpu.CompilerParams(vmem_limit_bytes=100 * 1024 * 1024),
    )(x3, w1, b1, g1, be1, rms3, w2bf, b2, g2, be2)


def kernel(int_feats, missing_mask, table, missing_emb, W1, b1, ln1_g, ln1_b,
           rms_s, W2, b2, ln2_g, ln2_b):
    offs = (jnp.arange(F, dtype=jnp.int32) * VOCAB)[None, :]
    idx = int_feats.astype(jnp.int32) + offs
    ridx = idx.reshape(B, NG, 2).transpose(1, 0, 2).reshape(-1)
    m = missing_mask.astype(jnp.int32).reshape(B, NG, 2)
    code = (jnp.arange(NG, dtype=jnp.int32) * 4)[None, :] + m[..., 0] + 2 * m[..., 1]
    cidx = code.T.reshape(-1)
    me0 = missing_emb[0::2]
    me1 = missing_emb[1::2]
    ctab = jnp.stack(
        [jnp.zeros_like(me0), me0, me1, me0 + me1], axis=1
    ).reshape(NG * 4, EMB) * 0.5

    pooled = _sc_pool(table, ctab, ridx, cidx)
    x3 = pooled.reshape(NG, B, EMB)

    out = _tc_call(
        x3, W1,
        b1.reshape(1, DM), ln1_g.reshape(1, DM), ln1_b.reshape(1, DM),
        rms_s.reshape(NG, 1, DM),
        W2.reshape(NG, DM, NT * DM).astype(jnp.bfloat16),
        b2.reshape(1, NT * DM), ln2_g.reshape(1, NT * DM),
        ln2_b.reshape(1, NT * DM),
    )
    return out.reshape(B, NT, DM)


# speedup vs baseline: 1.2465x; 1.0071x over previous
import functools

import jax
import jax.numpy as jnp
from jax import lax
from jax.experimental import pallas as pl
from jax.experimental.pallas import tpu as pltpu
from jax.experimental.pallas import tpu_sc as plsc

B = 1024
F = 26
VOCAB = 1000
EMB = 128
DM = 512
NG = 13
NT = 8

NW = 32
PAIRS = B * NG
PW = PAIRS // NW
CHP = 32
NCHUNK = PW // CHP


def _sc_pool(table, ctab, ridx, cidx):
    mesh = plsc.VectorSubcoreMesh(core_axis_name="c", subcore_axis_name="s")

    @functools.partial(
        pl.kernel,
        mesh=mesh,
        out_type=jax.ShapeDtypeStruct((PAIRS, EMB), jnp.float32),
        scratch_types=[
            pltpu.VMEM((2, 2 * CHP), jnp.int32),
            pltpu.VMEM((2, CHP), jnp.int32),
            pltpu.VMEM((2, 2 * CHP, EMB), jnp.float32),
            pltpu.VMEM((2, CHP, EMB), jnp.float32),
            pltpu.VMEM((2, CHP, EMB), jnp.float32),
            pltpu.SemaphoreType.DMA,
            pltpu.SemaphoreType.DMA,
            pltpu.SemaphoreType.DMA,
            pltpu.SemaphoreType.DMA,
            pltpu.SemaphoreType.DMA,
            pltpu.SemaphoreType.DMA,
        ],
    )
    def k(table_h, ctab_h, ridx_h, cidx_h, out_h,
          ridx_v, cidx_v, rows_v, crows_v, out_v,
          semr0, semr1, semc0, semc1, semo0, semo1):
        wid = lax.axis_index("s") * 2 + lax.axis_index("c")
        semr = (semr0, semr1)
        semc = (semc0, semc1)
        semo = (semo0, semo1)

        def stage(ci):
            b = ci % 2
            pbase = wid * PW + ci * CHP
            pltpu.sync_copy(ridx_h.at[pl.ds(pbase * 2, 2 * CHP)], ridx_v.at[b])
            pltpu.sync_copy(cidx_h.at[pl.ds(pbase, CHP)], cidx_v.at[b])
            cpr = pltpu.async_copy(table_h.at[ridx_v.at[b]], rows_v.at[b], semr[b])
            cpc = pltpu.async_copy(ctab_h.at[cidx_v.at[b]], crows_v.at[b], semc[b])
            return cpr, cpc

        inflight = stage(0)
        for ci in range(NCHUNK):
            b = ci % 2
            cur = inflight
            if ci + 1 < NCHUNK:
                inflight = stage(ci + 1)
            cur[0].wait()
            cur[1].wait()
            if ci >= 2:
                outcp[b].wait()

            def pair(p, _2, _b=b):
                for cc in range(EMB // 16):
                    s = pl.ds(cc * 16, 16)
                    out_v[_b, p, s] = (
                        (rows_v[_b, 2 * p, s] + rows_v[_b, 2 * p + 1, s]) * 0.5
                        + crows_v[_b, p, s])
                return 0

            lax.fori_loop(0, CHP, pair, 0)
            pbase = wid * PW + ci * CHP
            cp = pltpu.async_copy(out_v.at[b], out_h.at[pl.ds(pbase, CHP)], semo[b])
            if ci == 0:
                outcp = [cp, None]
            else:
                outcp[b] = cp
        outcp[(NCHUNK - 2) % 2].wait()
        outcp[(NCHUNK - 1) % 2].wait()

    return k(table, ctab, ridx, cidx)


BT = 128


def _silu(x):
    return x / (1.0 + jnp.exp(-x))


def _tc_body(x3, w1, b1, g1, be1, rms3, w2, b2, g2, be2, outr):
    hs = []
    ss = jnp.zeros((BT, 1), jnp.float32)
    w1v = w1[...]
    for g in range(NG):
        h = jnp.dot(x3[g].astype(jnp.bfloat16), w1v,
                    preferred_element_type=jnp.float32)
        h = _silu(h + b1[...])
        mu = jnp.mean(h, axis=1, keepdims=True)
        hc = h - mu
        var = jnp.mean(hc * hc, axis=1, keepdims=True)
        h = hc * lax.rsqrt(var + 1e-5) * g1[...] + be1[...]
        hs.append(h)
        ss = ss + jnp.sum(h * h, axis=1, keepdims=True)
    inv = lax.rsqrt(ss * (1.0 / (NG * DM)) + 1e-6)
    acc = jnp.zeros((BT, NT * DM), jnp.float32)
    for g in range(NG):
        hn = hs[g] * inv * rms3[g]
        acc = acc + jnp.dot(hn.astype(jnp.bfloat16), w2[g],
                            preferred_element_type=jnp.float32)
    y = _silu(acc + b2[...])
    mu = jnp.mean(y, axis=1, keepdims=True)
    yc = y - mu
    var = jnp.mean(yc * yc, axis=1, keepdims=True)
    outr[...] = yc * lax.rsqrt(var + 1e-5) * g2[...] + be2[...]


def _tc_call(x3, w1, b1, g1, be1, rms3, w2bf, b2, g2, be2):
    return pl.pallas_call(
        _tc_body,
        grid=(B // BT,),
        in_specs=[
            pl.BlockSpec((NG, BT, EMB), lambda i: (0, i, 0)),
            pl.BlockSpec((EMB, DM), lambda i: (0, 0)),
            pl.BlockSpec((1, DM), lambda i: (0, 0)),
            pl.BlockSpec((1, DM), lambda i: (0, 0)),
            pl.BlockSpec((1, DM), lambda i: (0, 0)),
            pl.BlockSpec((NG, 1, DM), lambda i: (0, 0, 0)),
            pl.BlockSpec((NG, DM, NT * DM), lambda i: (0, 0, 0)),
            pl.BlockSpec((1, NT * DM), lambda i: (0, 0)),
            pl.BlockSpec((1, NT * DM), lambda i: (0, 0)),
            pl.BlockSpec((1, NT * DM), lambda i: (0, 0)),
        ],
        out_specs=pl.BlockSpec((BT, NT * DM), lambda i: (i, 0)),
        out_shape=jax.ShapeDtypeStruct((B, NT * DM), jnp.float32),
        compiler_params=pltpu.CompilerParams(vmem_limit_bytes=100 * 1024 * 1024),
    )(x3, w1, b1, g1, be1, rms3, w2bf, b2, g2, be2)


def kernel(int_feats, missing_mask, table, missing_emb, W1, b1, ln1_g, ln1_b,
           rms_s, W2, b2, ln2_g, ln2_b):
    offs = (jnp.arange(F, dtype=jnp.int32) * VOCAB)[None, :]
    idx = int_feats.astype(jnp.int32) + offs
    ridx = idx.reshape(B, NG, 2).transpose(1, 0, 2).reshape(-1)
    m = missing_mask.astype(jnp.int32).reshape(B, NG, 2)
    code = (jnp.arange(NG, dtype=jnp.int32) * 4)[None, :] + m[..., 0] + 2 * m[..., 1]
    cidx = code.T.reshape(-1)
    me0 = missing_emb[0::2]
    me1 = missing_emb[1::2]
    ctab = jnp.stack(
        [jnp.zeros_like(me0), me0, me1, me0 + me1], axis=1
    ).reshape(NG * 4, EMB) * 0.5

    pooled = _sc_pool(table, ctab, ridx, cidx)
    x3 = pooled.reshape(NG, B, EMB)

    out = _tc_call(
        x3, W1.astype(jnp.bfloat16),
        b1.reshape(1, DM), ln1_g.reshape(1, DM), ln1_b.reshape(1, DM),
        rms_s.reshape(NG, 1, DM),
        W2.reshape(NG, DM, NT * DM).astype(jnp.bfloat16),
        b2.reshape(1, NT * DM), ln2_g.reshape(1, NT * DM),
        ln2_b.reshape(1, NT * DM),
    )
    return out.reshape(B, NT, DM)


# speedup vs baseline: 1.2705x; 1.0193x over previous
import functools

import jax
import jax.numpy as jnp
from jax import lax
from jax.experimental import pallas as pl
from jax.experimental.pallas import tpu as pltpu
from jax.experimental.pallas import tpu_sc as plsc

B = 1024
F = 26
VOCAB = 1000
EMB = 128
DM = 512
NG = 13
NT = 8

NW = 32
PAIRS = B * NG
PW = PAIRS // NW
CHP = 104
NCHUNK = PW // CHP


def _sc_pool(table, ctab, ridx, cidx):
    mesh = plsc.VectorSubcoreMesh(core_axis_name="c", subcore_axis_name="s")

    @functools.partial(
        pl.kernel,
        mesh=mesh,
        out_type=jax.ShapeDtypeStruct((PAIRS, EMB), jnp.float32),
        scratch_types=[
            pltpu.VMEM((2 * CHP,), jnp.int32),
            pltpu.VMEM((2 * CHP,), jnp.int32),
            pltpu.VMEM((CHP,), jnp.int32),
            pltpu.VMEM((CHP,), jnp.int32),
            pltpu.VMEM((2 * CHP, EMB), jnp.float32),
            pltpu.VMEM((2 * CHP, EMB), jnp.float32),
            pltpu.VMEM((CHP, EMB), jnp.float32),
            pltpu.VMEM((CHP, EMB), jnp.float32),
            pltpu.VMEM((CHP, EMB), jnp.float32),
            pltpu.VMEM((CHP, EMB), jnp.float32),
            pltpu.SemaphoreType.DMA,
            pltpu.SemaphoreType.DMA,
            pltpu.SemaphoreType.DMA,
            pltpu.SemaphoreType.DMA,
            pltpu.SemaphoreType.DMA,
            pltpu.SemaphoreType.DMA,
        ],
    )
    def k(table_h, ctab_h, ridx_h, cidx_h, out_h,
          ridx_v0, ridx_v1, cidx_v0, cidx_v1, rows_v0, rows_v1,
          crows_v0, crows_v1, out_v0, out_v1,
          semr0, semr1, semc0, semc1, semo0, semo1):
        wid = lax.axis_index("s") * 2 + lax.axis_index("c")
        ridx_v = (ridx_v0, ridx_v1)
        cidx_v = (cidx_v0, cidx_v1)
        rows_v = (rows_v0, rows_v1)
        crows_v = (crows_v0, crows_v1)
        out_v = (out_v0, out_v1)
        semr = (semr0, semr1)
        semc = (semc0, semc1)
        semo = (semo0, semo1)

        def stage(ci):
            b = ci % 2
            pbase = wid * PW + ci * CHP
            pltpu.sync_copy(ridx_h.at[pl.ds(pbase * 2, 2 * CHP)], ridx_v[b])
            pltpu.sync_copy(cidx_h.at[pl.ds(pbase, CHP)], cidx_v[b])
            cpr0 = pltpu.async_copy(table_h.at[ridx_v[b].at[pl.ds(0, CHP)]],
                                    rows_v[b].at[pl.ds(0, CHP)], semr[b])
            cpr1 = pltpu.async_copy(table_h.at[ridx_v[b].at[pl.ds(CHP, CHP)]],
                                    rows_v[b].at[pl.ds(CHP, CHP)], semr[b])
            cpc = pltpu.async_copy(ctab_h.at[cidx_v[b]], crows_v[b], semc[b])
            return cpr0, cpr1, cpc

        inflight = stage(0)
        for ci in range(NCHUNK):
            b = ci % 2
            cur = inflight
            if ci + 1 < NCHUNK:
                inflight = stage(ci + 1)
            cur[0].wait()
            cur[1].wait()
            cur[2].wait()
            if ci >= 2:
                outcp[b].wait()

            def pair(p, _2, _b=b):
                for cc in range(EMB // 16):
                    s = pl.ds(cc * 16, 16)
                    out_v[_b][p, s] = (
                        (rows_v[_b][2 * p, s] + rows_v[_b][2 * p + 1, s]) * 0.5
                        + crows_v[_b][p, s])
                return 0

            lax.fori_loop(0, CHP, pair, 0)
            pbase = wid * PW + ci * CHP
            cp = pltpu.async_copy(out_v[b], out_h.at[pl.ds(pbase, CHP)], semo[b])
            if ci == 0:
                outcp = [cp, None]
            else:
                outcp[b] = cp
        outcp[(NCHUNK - 2) % 2].wait()
        outcp[(NCHUNK - 1) % 2].wait()

    return k(table, ctab, ridx, cidx)


BT = 128


def _silu(x):
    return x / (1.0 + jnp.exp(-x))


def _tc_body(x3, w1, b1, g1, be1, rms3, w2, b2, g2, be2, outr, hbuf):
    ss = jnp.zeros((BT, 1), jnp.float32)
    w1v = w1[...]
    for g in range(NG):
        h = jnp.dot(x3[g].astype(jnp.bfloat16), w1v,
                    preferred_element_type=jnp.float32)
        h = _silu(h + b1[...])
        mu = jnp.mean(h, axis=1, keepdims=True)
        hc = h - mu
        var = jnp.mean(hc * hc, axis=1, keepdims=True)
        h = hc * lax.rsqrt(var + 1e-5) * g1[...] + be1[...]
        ss = ss + jnp.sum(h * h, axis=1, keepdims=True)
        hbuf[:, pl.ds(g * DM, DM)] = (h * rms3[g]).astype(jnp.bfloat16)
    inv = lax.rsqrt(ss * (1.0 / (NG * DM)) + 1e-6)
    y = jnp.dot(hbuf[...], w2[...], preferred_element_type=jnp.float32) * inv
    y = _silu(y + b2[...])
    mu = jnp.mean(y, axis=1, keepdims=True)
    yc = y - mu
    var = jnp.mean(yc * yc, axis=1, keepdims=True)
    outr[...] = yc * lax.rsqrt(var + 1e-5) * g2[...] + be2[...]


def _tc_call(x3, w1, b1, g1, be1, rms3, w2bf, b2, g2, be2):
    return pl.pallas_call(
        _tc_body,
        grid=(B // BT,),
        in_specs=[
            pl.BlockSpec((NG, BT, EMB), lambda i: (0, i, 0)),
            pl.BlockSpec((EMB, DM), lambda i: (0, 0)),
            pl.BlockSpec((1, DM), lambda i: (0, 0)),
            pl.BlockSpec((1, DM), lambda i: (0, 0)),
            pl.BlockSpec((1, DM), lambda i: (0, 0)),
            pl.BlockSpec((NG, 1, DM), lambda i: (0, 0, 0)),
            pl.BlockSpec((NG * DM, NT * DM), lambda i: (0, 0)),
            pl.BlockSpec((1, NT * DM), lambda i: (0, 0)),
            pl.BlockSpec((1, NT * DM), lambda i: (0, 0)),
            pl.BlockSpec((1, NT * DM), lambda i: (0, 0)),
        ],
        out_specs=pl.BlockSpec((BT, NT * DM), lambda i: (i, 0)),
        out_shape=jax.ShapeDtypeStruct((B, NT * DM), jnp.float32),
        scratch_shapes=[pltpu.VMEM((BT, NG * DM), jnp.bfloat16)],
        compiler_params=pltpu.CompilerParams(vmem_limit_bytes=100 * 1024 * 1024),
    )(x3, w1, b1, g1, be1, rms3, w2bf, b2, g2, be2)


def kernel(int_feats, missing_mask, table, missing_emb, W1, b1, ln1_g, ln1_b,
           rms_s, W2, b2, ln2_g, ln2_b):
    offs = (jnp.arange(F, dtype=jnp.int32) * VOCAB)[None, :]
    idx = int_feats.astype(jnp.int32) + offs
    ridx = idx.reshape(B, NG, 2).transpose(1, 0, 2).reshape(-1)
    m = missing_mask.astype(jnp.int32).reshape(B, NG, 2)
    code = (jnp.arange(NG, dtype=jnp.int32) * 4)[None, :] + m[..., 0] + 2 * m[..., 1]
    cidx = code.T.reshape(-1)
    me0 = missing_emb[0::2]
    me1 = missing_emb[1::2]
    ctab = jnp.stack(
        [jnp.zeros_like(me0), me0, me1, me0 + me1], axis=1
    ).reshape(NG * 4, EMB) * 0.5

    pooled = _sc_pool(table, ctab, ridx, cidx)
    x3 = pooled.reshape(NG, B, EMB)

    out = _tc_call(
        x3, W1.astype(jnp.bfloat16),
        b1.reshape(1, DM), ln1_g.reshape(1, DM), ln1_b.reshape(1, DM),
        rms_s.reshape(NG, 1, DM),
        W2.astype(jnp.bfloat16),
        b2.reshape(1, NT * DM), ln2_g.reshape(1, NT * DM),
        ln2_b.reshape(1, NT * DM),
    )
    return out.reshape(B, NT, DM)


# speedup vs baseline: 1.4417x; 1.1347x over previous
import functools

import jax
import jax.numpy as jnp
from jax import lax
from jax.experimental import pallas as pl
from jax.experimental.pallas import tpu as pltpu
from jax.experimental.pallas import tpu_sc as plsc

B = 1024
F = 26
VOCAB = 1000
EMB = 128
DM = 512
NG = 13
NT = 8

NW = 32
PAIRS = B * NG
NROWS = 2 * PAIRS
RW = NROWS // NW
CR = 104
NCHUNK = RW // CR


def _sc_gather(table, allidx):
    mesh = plsc.VectorSubcoreMesh(core_axis_name="c", subcore_axis_name="s")

    @functools.partial(
        pl.kernel,
        mesh=mesh,
        out_type=jax.ShapeDtypeStruct((NROWS, EMB), jnp.float32),
        scratch_types=[
            pltpu.VMEM((CR,), jnp.int32),
            pltpu.VMEM((CR,), jnp.int32),
            pltpu.VMEM((CR, EMB), jnp.float32),
            pltpu.VMEM((CR, EMB), jnp.float32),
            pltpu.SemaphoreType.DMA,
            pltpu.SemaphoreType.DMA,
            pltpu.SemaphoreType.DMA,
            pltpu.SemaphoreType.DMA,
        ],
    )
    def k(table_h, idx_h, out_h, idx_v0, idx_v1, buf_v0, buf_v1,
          semg0, semg1, semo0, semo1):
        wid = lax.axis_index("s") * 2 + lax.axis_index("c")
        idx_v = (idx_v0, idx_v1)
        buf_v = (buf_v0, buf_v1)
        semg = (semg0, semg1)
        semo = (semo0, semo1)

        def stage(ci):
            b = ci % 2
            rbase = wid * RW + ci * CR
            pltpu.sync_copy(idx_h.at[pl.ds(rbase, CR)], idx_v[b])
            return pltpu.async_copy(table_h.at[idx_v[b]], buf_v[b], semg[b])

        gcp = stage(0)
        outcp = [None, None]
        for ci in range(NCHUNK):
            b = ci % 2
            cur = gcp
            if ci + 1 < NCHUNK:
                nb = (ci + 1) % 2
                if outcp[nb] is not None:
                    outcp[nb].wait()
                gcp = stage(ci + 1)
            cur.wait()
            rbase = wid * RW + ci * CR
            outcp[b] = pltpu.async_copy(buf_v[b], out_h.at[pl.ds(rbase, CR)],
                                        semo[b])
        if outcp[NCHUNK % 2] is not None:
            outcp[NCHUNK % 2].wait()
        outcp[(NCHUNK - 1) % 2].wait()

    return k(table, allidx)


BT = 128


def _silu(x):
    return x / (1.0 + jnp.exp(-x))


def _tc_body(x4, mf, me2, w1, b1, g1, be1, rms3, w2, b2, g2, be2, outr, hbuf):
    ss = jnp.zeros((BT, 1), jnp.float32)
    w1v = w1[...]
    miss = jnp.dot(mf[...], me2[...], preferred_element_type=jnp.float32)
    for g in range(NG):
        xg = ((x4[0, g] + x4[1, g]).astype(jnp.float32) * 0.5
              + miss[:, g * EMB:(g + 1) * EMB])
        h = jnp.dot(xg.astype(jnp.bfloat16), w1v,
                    preferred_element_type=jnp.float32)
        h = _silu(h + b1[...])
        mu = jnp.mean(h, axis=1, keepdims=True)
        hc = h - mu
        var = jnp.mean(hc * hc, axis=1, keepdims=True)
        h = hc * lax.rsqrt(var + 1e-5) * g1[...] + be1[...]
        ss = ss + jnp.sum(h * h, axis=1, keepdims=True)
        hbuf[:, pl.ds(g * DM, DM)] = (h * rms3[g]).astype(jnp.bfloat16)
    inv = lax.rsqrt(ss * (1.0 / (NG * DM)) + 1e-6)
    y = jnp.dot(hbuf[...], w2[...], preferred_element_type=jnp.float32) * inv
    y = _silu(y + b2[...])
    mu = jnp.mean(y, axis=1, keepdims=True)
    yc = y - mu
    var = jnp.mean(yc * yc, axis=1, keepdims=True)
    outr[...] = yc * lax.rsqrt(var + 1e-5) * g2[...] + be2[...]


def _tc_call(x4, mf, me2, w1, b1, g1, be1, rms3, w2bf, b2, g2, be2):
    return pl.pallas_call(
        _tc_body,
        grid=(B // BT,),
        in_specs=[
            pl.BlockSpec((2, NG, BT, EMB), lambda i: (0, 0, i, 0)),
            pl.BlockSpec((BT, F), lambda i: (i, 0)),
            pl.BlockSpec((F, NG * EMB), lambda i: (0, 0)),
            pl.BlockSpec((EMB, DM), lambda i: (0, 0)),
            pl.BlockSpec((1, DM), lambda i: (0, 0)),
            pl.BlockSpec((1, DM), lambda i: (0, 0)),
            pl.BlockSpec((1, DM), lambda i: (0, 0)),
            pl.BlockSpec((NG, 1, DM), lambda i: (0, 0, 0)),
            pl.BlockSpec((NG * DM, NT * DM), lambda i: (0, 0)),
            pl.BlockSpec((1, NT * DM), lambda i: (0, 0)),
            pl.BlockSpec((1, NT * DM), lambda i: (0, 0)),
            pl.BlockSpec((1, NT * DM), lambda i: (0, 0)),
        ],
        out_specs=pl.BlockSpec((BT, NT * DM), lambda i: (i, 0)),
        out_shape=jax.ShapeDtypeStruct((B, NT * DM), jnp.float32),
        scratch_shapes=[pltpu.VMEM((BT, NG * DM), jnp.bfloat16)],
        compiler_params=pltpu.CompilerParams(vmem_limit_bytes=100 * 1024 * 1024),
    )(x4, mf, me2, w1, b1, g1, be1, rms3, w2bf, b2, g2, be2)


def kernel(int_feats, missing_mask, table, missing_emb, W1, b1, ln1_g, ln1_b,
           rms_s, W2, b2, ln2_g, ln2_b):
    offs = (jnp.arange(F, dtype=jnp.int32) * VOCAB)[None, :]
    idx = int_feats.astype(jnp.int32) + offs
    allidx = jnp.concatenate(
        [idx[:, 0::2].T.reshape(-1), idx[:, 1::2].T.reshape(-1)])
    mf = missing_mask.astype(jnp.float32)
    onehot = (jnp.arange(NG)[None, :] == (jnp.arange(F) // 2)[:, None])
    me2 = (onehot.astype(jnp.float32)[:, :, None]
           * missing_emb[:, None, :]).reshape(F, NG * EMB) * 0.5

    rows = _sc_gather(table, allidx)
    x4 = rows.reshape(2, NG, B, EMB).astype(jnp.bfloat16)

    out = _tc_call(
        x4, mf, me2, W1.astype(jnp.bfloat16),
        b1.reshape(1, DM), ln1_g.reshape(1, DM), ln1_b.reshape(1, DM),
        rms_s.reshape(NG, 1, DM),
        W2.astype(jnp.bfloat16),
        b2.reshape(1, NT * DM), ln2_g.reshape(1, NT * DM),
        ln2_b.reshape(1, NT * DM),
    )
    return out.reshape(B, NT, DM)
